# baseline (speedup 1.0000x reference)
import functools

import jax
import jax.numpy as jnp
from jax import lax
from jax.experimental import pallas as pl
from jax.experimental.pallas import tpu as pltpu
from jax.experimental.pallas import tpu_sc as plsc

E = 64
TOPK = 2
D = 1024
DFF = 512
T = 2048

ROWS = 128
NB = (T * TOPK) // ROWS + E
PAD = NB * ROWS

NC, NS = 2, 16
NW = NC * NS
GCH = 64
G_CHUNKS = PAD // (NW * GCH)
TPW = T // NW

_ROUTER_TILE = 256


def _router_body(x_ref, gw_ref, b_ref, idx_ref, w_ref):
    x = x_ref[...]
    gw = gw_ref[...]
    logits = lax.dot_general(x, gw, (((1,), (1,)), ((), ())),
                             preferred_element_type=jnp.float32)
    scores = jax.nn.sigmoid(logits)
    biased = scores + b_ref[...]
    cols = lax.broadcasted_iota(jnp.int32, biased.shape, 1)
    m1 = jnp.max(biased, axis=1, keepdims=True)
    i1 = jnp.min(jnp.where(biased == m1, cols, E), axis=1, keepdims=True)
    oh1 = cols == i1
    s1 = jnp.sum(jnp.where(oh1, scores, 0.0), axis=1, keepdims=True)
    masked = jnp.where(oh1, -jnp.inf, biased)
    m2 = jnp.max(masked, axis=1, keepdims=True)
    i2 = jnp.min(jnp.where(masked == m2, cols, E), axis=1, keepdims=True)
    s2 = jnp.sum(jnp.where(cols == i2, scores, 0.0), axis=1, keepdims=True)
    tot = s1 + s2
    idx_ref[...] = jnp.concatenate([i1, i2], axis=1)
    w_ref[...] = jnp.concatenate([s1 / tot, s2 / tot], axis=1)


def _router(x, gate_w, expert_bias):
    return pl.pallas_call(
        _router_body,
        grid=(T // _ROUTER_TILE,),
        in_specs=[
            pl.BlockSpec((_ROUTER_TILE, D), lambda i: (i, 0)),
            pl.BlockSpec((E, D), lambda i: (0, 0)),
            pl.BlockSpec((1, E), lambda i: (0, 0)),
        ],
        out_specs=[
            pl.BlockSpec((_ROUTER_TILE, TOPK), lambda i: (i, 0)),
            pl.BlockSpec((_ROUTER_TILE, TOPK), lambda i: (i, 0)),
        ],
        out_shape=[
            jax.ShapeDtypeStruct((T, TOPK), jnp.int32),
            jax.ShapeDtypeStruct((T, TOPK), jnp.float32),
        ],
    )(x, gate_w, expert_bias.reshape(1, E))


def _dispatch_maps(topk_idx):
    eq = topk_idx.reshape(-1).astype(jnp.int32)
    oh = eq[:, None] == jnp.arange(E, dtype=jnp.int32)[None, :]
    ranks_all = jnp.cumsum(oh.astype(jnp.int32), axis=0) - 1
    rank = jnp.sum(jnp.where(oh, ranks_all, 0), axis=1)
    counts = jnp.sum(oh.astype(jnp.int32), axis=0)
    nb = (counts + ROWS - 1) // ROWS
    pcount = nb * ROWS
    poff = jnp.concatenate([jnp.zeros((1,), jnp.int32),
                            jnp.cumsum(pcount)[:-1].astype(jnp.int32)])
    ppos = poff[eq] + rank
    cnb = jnp.cumsum(nb).astype(jnp.int32)
    total_nb = cnb[-1]
    blk = jnp.arange(NB, dtype=jnp.int32)
    be_raw = jnp.clip(jnp.searchsorted(cnb, blk, side='right'), 0, E - 1)
    be_raw = be_raw.astype(jnp.int32)
    br_raw = poff[be_raw] // ROWS + (blk - (cnb[be_raw] - nb[be_raw]))
    valid = blk < total_nb
    be = jnp.where(valid, be_raw, be_raw[total_nb - 1])
    br = jnp.where(valid, br_raw, br_raw[total_nb - 1])
    return be, br, ppos


def _dispatch_body(x_hbm, pos_hbm, xs_hbm, idx_v, rows_v, sem):
    wid = lax.axis_index("s") * NC + lax.axis_index("c")
    pltpu.sync_copy(pos_hbm.at[wid], idx_v)
    pltpu.sync_copy(x_hbm.at[pl.ds(wid * TPW, TPW)], rows_v)
    pltpu.async_copy(rows_v, xs_hbm.at[idx_v.at[0]], sem).wait()
    pltpu.async_copy(rows_v, xs_hbm.at[idx_v.at[1]], sem).wait()


_sc_dispatch = functools.partial(
    pl.kernel,
    mesh=plsc.VectorSubcoreMesh(core_axis_name="c", subcore_axis_name="s"),
    out_type=jax.ShapeDtypeStruct((PAD, D), jnp.float32),
    scratch_types=[
        pltpu.VMEM((TOPK, TPW), jnp.int32),
        pltpu.VMEM((TPW, D), jnp.float32),
        pltpu.SemaphoreType.DMA,
    ],
)(_dispatch_body)


def _make_sc_gather(n_rows):
    chunks = n_rows // (NW * GCH)

    def body(src_hbm, gidx_hbm, out_hbm, idx_v, rows_v, sem):
        wid = lax.axis_index("s") * NC + lax.axis_index("c")
        pltpu.sync_copy(gidx_hbm.at[pl.ds(wid * chunks, chunks)], idx_v)
        base = wid * (chunks * GCH)
        for c in range(chunks):
            pltpu.async_copy(src_hbm.at[idx_v.at[c]], rows_v, sem).wait()
            pltpu.sync_copy(rows_v, out_hbm.at[pl.ds(base + c * GCH, GCH)])

    return functools.partial(
        pl.kernel,
        mesh=plsc.VectorSubcoreMesh(core_axis_name="c", subcore_axis_name="s"),
        out_type=jax.ShapeDtypeStruct((n_rows, D), jnp.float32),
        scratch_types=[
            pltpu.VMEM((chunks, GCH), jnp.int32),
            pltpu.VMEM((GCH, D), jnp.float32),
            pltpu.SemaphoreType.DMA,
        ],
    )(body)


_sc_gather_pairs = _make_sc_gather(T * TOPK)


def _mlp_body(be_ref, br_ref, xs_ref, w1_ref, w3_ref, w2_ref, out_ref):
    del be_ref, br_ref
    x = xs_ref[...]
    a = jnp.dot(x, w1_ref[0], preferred_element_type=jnp.float32)
    g = a * jax.nn.sigmoid(a) * jnp.dot(x, w3_ref[0],
                                        preferred_element_type=jnp.float32)
    out_ref[...] = jnp.dot(g, w2_ref[0], preferred_element_type=jnp.float32)


def _grouped_mlp(be, br, xs, w1, w3, w2):
    grid_spec = pltpu.PrefetchScalarGridSpec(
        num_scalar_prefetch=2,
        grid=(NB,),
        in_specs=[
            pl.BlockSpec((ROWS, D), lambda i, be, br: (i, 0)),
            pl.BlockSpec((1, D, DFF), lambda i, be, br: (i // 2, 0, 0)),
            pl.BlockSpec((1, D, DFF), lambda i, be, br: (i // 2, 0, 0)),
            pl.BlockSpec((1, DFF, D), lambda i, be, br: (i // 2, 0, 0)),
        ],
        out_specs=pl.BlockSpec((ROWS, D), lambda i, be, br: (i, 0)),
    )
    return pl.pallas_call(
        _mlp_body,
        grid_spec=grid_spec,
        out_shape=jax.ShapeDtypeStruct((PAD, D), jnp.float32),
        compiler_params=pltpu.CompilerParams(
            dimension_semantics=("arbitrary",)),
    )(be, br, xs, w1, w3, w2)


def _add_body(a_ref, b_ref, w_ref, o_ref):
    w = w_ref[...]
    o_ref[...] = a_ref[...] * w[:, 0:1] + b_ref[...] * w[:, 1:2]


def _pair_add(ypair, topk_w):
    tile = 256
    return pl.pallas_call(
        _add_body,
        grid=(T // tile,),
        in_specs=[
            pl.BlockSpec((tile, D), lambda i: (i, 0)),
            pl.BlockSpec((tile, D), lambda i: (i + T // tile, 0)),
            pl.BlockSpec((tile, TOPK), lambda i: (i, 0)),
        ],
        out_specs=pl.BlockSpec((tile, D), lambda i: (i, 0)),
        out_shape=jax.ShapeDtypeStruct((T, D), jnp.float32),
    )(ypair, ypair, topk_w)


def kernel(hidden_states, gate_w, expert_bias, w1, w3, w2):
    topk_idx, topk_w = _router(hidden_states, gate_w, expert_bias)
    be, br, ppos = _dispatch_maps(topk_idx)
    pos = ppos.reshape(NW, TPW, TOPK).transpose(0, 2, 1)
    xs = _sc_dispatch(hidden_states, pos)
    ysc = _grouped_mlp(be, br, xs, w1, w3, w2)
    pidx = jnp.concatenate([ppos[0::2], ppos[1::2]])
    ypair = _sc_gather_pairs(ysc, pidx.reshape(-1, GCH))
    return _pair_add(ypair, topk_w)


# speedup vs baseline: 1.0729x; 1.0729x over previous
import functools

import jax
import jax.numpy as jnp
from jax import lax
from jax.experimental import pallas as pl
from jax.experimental.pallas import tpu as pltpu
from jax.experimental.pallas import tpu_sc as plsc

E = 64
TOPK = 2
D = 1024
DFF = 512
T = 2048

ROWS = 128
NB = (T * TOPK) // ROWS + E
PAD = NB * ROWS

NC, NS = 2, 16
NW = NC * NS
GCH = 64
G_CHUNKS = PAD // (NW * GCH)
TPW = T // NW

_ROUTER_TILE = 256


def _router_body(x_ref, gw_ref, b_ref, idx_ref, w_ref):
    x = x_ref[...]
    gw = gw_ref[...]
    logits = lax.dot_general(x, gw, (((1,), (1,)), ((), ())),
                             preferred_element_type=jnp.float32)
    scores = jax.nn.sigmoid(logits)
    biased = scores + b_ref[...]
    cols = lax.broadcasted_iota(jnp.int32, biased.shape, 1)
    m1 = jnp.max(biased, axis=1, keepdims=True)
    i1 = jnp.min(jnp.where(biased == m1, cols, E), axis=1, keepdims=True)
    oh1 = cols == i1
    s1 = jnp.sum(jnp.where(oh1, scores, 0.0), axis=1, keepdims=True)
    masked = jnp.where(oh1, -jnp.inf, biased)
    m2 = jnp.max(masked, axis=1, keepdims=True)
    i2 = jnp.min(jnp.where(masked == m2, cols, E), axis=1, keepdims=True)
    s2 = jnp.sum(jnp.where(cols == i2, scores, 0.0), axis=1, keepdims=True)
    tot = s1 + s2
    idx_ref[...] = jnp.concatenate([i1, i2], axis=1)
    w_ref[...] = jnp.concatenate([s1 / tot, s2 / tot], axis=1)


def _router(x, gate_w, expert_bias):
    return pl.pallas_call(
        _router_body,
        grid=(T // _ROUTER_TILE,),
        in_specs=[
            pl.BlockSpec((_ROUTER_TILE, D), lambda i: (i, 0)),
            pl.BlockSpec((E, D), lambda i: (0, 0)),
            pl.BlockSpec((1, E), lambda i: (0, 0)),
        ],
        out_specs=[
            pl.BlockSpec((_ROUTER_TILE, TOPK), lambda i: (i, 0)),
            pl.BlockSpec((_ROUTER_TILE, TOPK), lambda i: (i, 0)),
        ],
        out_shape=[
            jax.ShapeDtypeStruct((T, TOPK), jnp.int32),
            jax.ShapeDtypeStruct((T, TOPK), jnp.float32),
        ],
    )(x, gate_w, expert_bias.reshape(1, E))


def _dispatch_maps(topk_idx):
    eq = topk_idx.reshape(-1).astype(jnp.int32)
    oh = eq[:, None] == jnp.arange(E, dtype=jnp.int32)[None, :]
    ranks_all = jnp.cumsum(oh.astype(jnp.int32), axis=0) - 1
    rank = jnp.sum(jnp.where(oh, ranks_all, 0), axis=1)
    counts = jnp.sum(oh.astype(jnp.int32), axis=0)
    nb = (counts + ROWS - 1) // ROWS
    pcount = nb * ROWS
    poff = jnp.concatenate([jnp.zeros((1,), jnp.int32),
                            jnp.cumsum(pcount)[:-1].astype(jnp.int32)])
    ppos = poff[eq] + rank
    cnb = jnp.cumsum(nb).astype(jnp.int32)
    total_nb = cnb[-1]
    blk = jnp.arange(NB, dtype=jnp.int32)
    be_raw = jnp.clip(jnp.searchsorted(cnb, blk, side='right'), 0, E - 1)
    be_raw = be_raw.astype(jnp.int32)
    br_raw = poff[be_raw] // ROWS + (blk - (cnb[be_raw] - nb[be_raw]))
    valid = blk < total_nb
    be = jnp.where(valid, be_raw, be_raw[total_nb - 1])
    br = jnp.where(valid, br_raw, br_raw[total_nb - 1])
    return be, br, ppos


def _dispatch_body(x_hbm, pos_hbm, xs_hbm, idx_v, rows_v, sem):
    wid = lax.axis_index("s") * NC + lax.axis_index("c")
    pltpu.sync_copy(pos_hbm.at[wid], idx_v)
    pltpu.sync_copy(x_hbm.at[pl.ds(wid * TPW, TPW)], rows_v)
    pltpu.async_copy(rows_v, xs_hbm.at[idx_v.at[0]], sem).wait()
    pltpu.async_copy(rows_v, xs_hbm.at[idx_v.at[1]], sem).wait()


_sc_dispatch = functools.partial(
    pl.kernel,
    mesh=plsc.VectorSubcoreMesh(core_axis_name="c", subcore_axis_name="s"),
    out_type=jax.ShapeDtypeStruct((PAD, D), jnp.float32),
    scratch_types=[
        pltpu.VMEM((TOPK, TPW), jnp.int32),
        pltpu.VMEM((TPW, D), jnp.float32),
        pltpu.SemaphoreType.DMA,
    ],
)(_dispatch_body)


def _make_sc_gather(n_rows):
    chunks = n_rows // (NW * GCH)

    def body(src_hbm, gidx_hbm, out_hbm, idx_v, rows_v, sem):
        wid = lax.axis_index("s") * NC + lax.axis_index("c")
        pltpu.sync_copy(gidx_hbm.at[pl.ds(wid * chunks, chunks)], idx_v)
        base = wid * (chunks * GCH)
        for c in range(chunks):
            pltpu.async_copy(src_hbm.at[idx_v.at[c]], rows_v, sem).wait()
            pltpu.sync_copy(rows_v, out_hbm.at[pl.ds(base + c * GCH, GCH)])

    return functools.partial(
        pl.kernel,
        mesh=plsc.VectorSubcoreMesh(core_axis_name="c", subcore_axis_name="s"),
        out_type=jax.ShapeDtypeStruct((n_rows, D), jnp.float32),
        scratch_types=[
            pltpu.VMEM((chunks, GCH), jnp.int32),
            pltpu.VMEM((GCH, D), jnp.float32),
            pltpu.SemaphoreType.DMA,
        ],
    )(body)


_sc_gather_pairs = _make_sc_gather(T * TOPK)


def _mlp_body(be_ref, br_ref, xs_ref, w1_ref, w3_ref, w2_ref, out_ref):
    del be_ref, br_ref
    x = xs_ref[...].astype(jnp.bfloat16)
    a = jnp.dot(x, w1_ref[0].astype(jnp.bfloat16),
                preferred_element_type=jnp.float32)
    g = a * jax.nn.sigmoid(a) * jnp.dot(x, w3_ref[0].astype(jnp.bfloat16),
                                        preferred_element_type=jnp.float32)
    out_ref[...] = jnp.dot(g.astype(jnp.bfloat16),
                           w2_ref[0].astype(jnp.bfloat16),
                           preferred_element_type=jnp.float32)


def _grouped_mlp(be, br, xs, w1, w3, w2):
    grid_spec = pltpu.PrefetchScalarGridSpec(
        num_scalar_prefetch=2,
        grid=(NB,),
        in_specs=[
            pl.BlockSpec((ROWS, D), lambda i, be, br: (i, 0)),
            pl.BlockSpec((1, D, DFF), lambda i, be, br: (be[i], 0, 0)),
            pl.BlockSpec((1, D, DFF), lambda i, be, br: (be[i], 0, 0)),
            pl.BlockSpec((1, DFF, D), lambda i, be, br: (be[i], 0, 0)),
        ],
        out_specs=pl.BlockSpec((ROWS, D), lambda i, be, br: (i, 0)),
    )
    return pl.pallas_call(
        _mlp_body,
        grid_spec=grid_spec,
        out_shape=jax.ShapeDtypeStruct((PAD, D), jnp.float32),
        compiler_params=pltpu.CompilerParams(
            dimension_semantics=("arbitrary",)),
    )(be, br, xs, w1, w3, w2)


def _add_body(a_ref, b_ref, w_ref, o_ref):
    w = w_ref[...]
    o_ref[...] = a_ref[...] * w[:, 0:1] + b_ref[...] * w[:, 1:2]


def _pair_add(ypair, topk_w):
    tile = 256
    return pl.pallas_call(
        _add_body,
        grid=(T // tile,),
        in_specs=[
            pl.BlockSpec((tile, D), lambda i: (i, 0)),
            pl.BlockSpec((tile, D), lambda i: (i + T // tile, 0)),
            pl.BlockSpec((tile, TOPK), lambda i: (i, 0)),
        ],
        out_specs=pl.BlockSpec((tile, D), lambda i: (i, 0)),
        out_shape=jax.ShapeDtypeStruct((T, D), jnp.float32),
    )(ypair, ypair, topk_w)


def kernel(hidden_states, gate_w, expert_bias, w1, w3, w2):
    topk_idx, topk_w = _router(hidden_states, gate_w, expert_bias)
    be, br, ppos = _dispatch_maps(topk_idx)
    pos = ppos.reshape(NW, TPW, TOPK).transpose(0, 2, 1)
    xs = _sc_dispatch(hidden_states, pos)
    ysc = _grouped_mlp(be, br, xs, w1, w3, w2)
    pidx = jnp.concatenate([ppos[0::2], ppos[1::2]])
    ypair = _sc_gather_pairs(ysc, pidx.reshape(-1, GCH))
    return _pair_add(ypair, topk_w)


# speedup vs baseline: 1.1228x; 1.0465x over previous
import functools

import jax
import jax.numpy as jnp
from jax import lax
from jax.experimental import pallas as pl
from jax.experimental.pallas import tpu as pltpu
from jax.experimental.pallas import tpu_sc as plsc

E = 64
TOPK = 2
D = 1024
DFF = 512
T = 2048

ROWS = 128
NB = (T * TOPK) // ROWS + E
PAD = NB * ROWS

NC, NS = 2, 16
NW = NC * NS
GCH = 64
G_CHUNKS = PAD // (NW * GCH)
TPW = T // NW

_ROUTER_TILE = 256


def _router_body(x_ref, gw_ref, b_ref, idx_ref, w_ref):
    x = x_ref[...]
    gw = gw_ref[...]
    logits = lax.dot_general(x, gw, (((1,), (1,)), ((), ())),
                             preferred_element_type=jnp.float32)
    scores = jax.nn.sigmoid(logits)
    biased = scores + b_ref[...]
    cols = lax.broadcasted_iota(jnp.int32, biased.shape, 1)
    m1 = jnp.max(biased, axis=1, keepdims=True)
    i1 = jnp.min(jnp.where(biased == m1, cols, E), axis=1, keepdims=True)
    oh1 = cols == i1
    s1 = jnp.sum(jnp.where(oh1, scores, 0.0), axis=1, keepdims=True)
    masked = jnp.where(oh1, -jnp.inf, biased)
    m2 = jnp.max(masked, axis=1, keepdims=True)
    i2 = jnp.min(jnp.where(masked == m2, cols, E), axis=1, keepdims=True)
    s2 = jnp.sum(jnp.where(cols == i2, scores, 0.0), axis=1, keepdims=True)
    tot = s1 + s2
    idx_ref[...] = jnp.concatenate([i1, i2], axis=1)
    w_ref[...] = jnp.concatenate([s1 / tot, s2 / tot], axis=1)


def _router(x, gate_w, expert_bias):
    return pl.pallas_call(
        _router_body,
        grid=(T // _ROUTER_TILE,),
        in_specs=[
            pl.BlockSpec((_ROUTER_TILE, D), lambda i: (i, 0)),
            pl.BlockSpec((E, D), lambda i: (0, 0)),
            pl.BlockSpec((1, E), lambda i: (0, 0)),
        ],
        out_specs=[
            pl.BlockSpec((_ROUTER_TILE, TOPK), lambda i: (i, 0)),
            pl.BlockSpec((_ROUTER_TILE, TOPK), lambda i: (i, 0)),
        ],
        out_shape=[
            jax.ShapeDtypeStruct((T, TOPK), jnp.int32),
            jax.ShapeDtypeStruct((T, TOPK), jnp.float32),
        ],
    )(x, gate_w, expert_bias.reshape(1, E))


def _dispatch_maps(topk_idx):
    eq = topk_idx.reshape(-1).astype(jnp.int32)
    oh = eq[:, None] == jnp.arange(E, dtype=jnp.int32)[None, :]
    ranks_all = jnp.cumsum(oh.astype(jnp.int32), axis=0) - 1
    rank = jnp.sum(jnp.where(oh, ranks_all, 0), axis=1)
    counts = jnp.sum(oh.astype(jnp.int32), axis=0)
    nb = (counts + ROWS - 1) // ROWS
    pcount = nb * ROWS
    poff = jnp.concatenate([jnp.zeros((1,), jnp.int32),
                            jnp.cumsum(pcount)[:-1].astype(jnp.int32)])
    ppos = poff[eq] + rank
    cnb = jnp.cumsum(nb).astype(jnp.int32)
    total_nb = cnb[-1]
    blk = jnp.arange(NB, dtype=jnp.int32)
    be_raw = jnp.clip(jnp.searchsorted(cnb, blk, side='right'), 0, E - 1)
    be_raw = be_raw.astype(jnp.int32)
    br_raw = poff[be_raw] // ROWS + (blk - (cnb[be_raw] - nb[be_raw]))
    valid = blk < total_nb
    be = jnp.where(valid, be_raw, be_raw[total_nb - 1])
    del br_raw
    return be, total_nb.reshape(1), ppos


def _dispatch_body(x_hbm, pos_hbm, xs_hbm, idx_v, rows_v, sem):
    wid = lax.axis_index("s") * NC + lax.axis_index("c")
    pltpu.sync_copy(pos_hbm.at[wid], idx_v)
    pltpu.sync_copy(x_hbm.at[pl.ds(wid * TPW, TPW)], rows_v)
    pltpu.async_copy(rows_v, xs_hbm.at[idx_v.at[0]], sem).wait()
    pltpu.async_copy(rows_v, xs_hbm.at[idx_v.at[1]], sem).wait()


_sc_dispatch = functools.partial(
    pl.kernel,
    mesh=plsc.VectorSubcoreMesh(core_axis_name="c", subcore_axis_name="s"),
    out_type=jax.ShapeDtypeStruct((PAD, D), jnp.float32),
    scratch_types=[
        pltpu.VMEM((TOPK, TPW), jnp.int32),
        pltpu.VMEM((TPW, D), jnp.float32),
        pltpu.SemaphoreType.DMA,
    ],
)(_dispatch_body)


def _make_sc_gather(n_rows):
    chunks = n_rows // (NW * GCH)

    def body(src_hbm, gidx_hbm, out_hbm, idx_v, rows_v, sem):
        wid = lax.axis_index("s") * NC + lax.axis_index("c")
        pltpu.sync_copy(gidx_hbm.at[pl.ds(wid * chunks, chunks)], idx_v)
        base = wid * (chunks * GCH)
        for c in range(chunks):
            pltpu.async_copy(src_hbm.at[idx_v.at[c]], rows_v, sem).wait()
            pltpu.sync_copy(rows_v, out_hbm.at[pl.ds(base + c * GCH, GCH)])

    return functools.partial(
        pl.kernel,
        mesh=plsc.VectorSubcoreMesh(core_axis_name="c", subcore_axis_name="s"),
        out_type=jax.ShapeDtypeStruct((n_rows, D), jnp.float32),
        scratch_types=[
            pltpu.VMEM((chunks, GCH), jnp.int32),
            pltpu.VMEM((GCH, D), jnp.float32),
            pltpu.SemaphoreType.DMA,
        ],
    )(body)


_sc_gather_pairs = _make_sc_gather(T * TOPK)


def _mlp_body(be_ref, tn_ref, xs_ref, w1_ref, w3_ref, w2_ref, out_ref):
    del be_ref

    @pl.when(pl.program_id(0) < tn_ref[0])
    def _():
        x = xs_ref[...]
        a = jnp.dot(x, w1_ref[0], preferred_element_type=jnp.float32)
        g = a * jax.nn.sigmoid(a) * jnp.dot(x, w3_ref[0],
                                            preferred_element_type=jnp.float32)
        out_ref[...] = jnp.dot(g, w2_ref[0], preferred_element_type=jnp.float32)


def _grouped_mlp(be, total_nb, xs, w1, w3, w2):
    grid_spec = pltpu.PrefetchScalarGridSpec(
        num_scalar_prefetch=2,
        grid=(NB,),
        in_specs=[
            pl.BlockSpec((ROWS, D), lambda i, be, tn: (i, 0)),
            pl.BlockSpec((1, D, DFF), lambda i, be, tn: (be[i], 0, 0)),
            pl.BlockSpec((1, D, DFF), lambda i, be, tn: (be[i], 0, 0)),
            pl.BlockSpec((1, DFF, D), lambda i, be, tn: (be[i], 0, 0)),
        ],
        out_specs=pl.BlockSpec((ROWS, D), lambda i, be, tn: (i, 0)),
    )
    return pl.pallas_call(
        _mlp_body,
        grid_spec=grid_spec,
        out_shape=jax.ShapeDtypeStruct((PAD, D), jnp.float32),
        compiler_params=pltpu.CompilerParams(
            dimension_semantics=("arbitrary",)),
    )(be, total_nb, xs, w1, w3, w2)


def _add_body(a_ref, b_ref, w_ref, o_ref):
    w = w_ref[...]
    o_ref[...] = a_ref[...] * w[:, 0:1] + b_ref[...] * w[:, 1:2]


def _pair_add(ypair, topk_w):
    tile = 256
    return pl.pallas_call(
        _add_body,
        grid=(T // tile,),
        in_specs=[
            pl.BlockSpec((tile, D), lambda i: (i, 0)),
            pl.BlockSpec((tile, D), lambda i: (i + T // tile, 0)),
            pl.BlockSpec((tile, TOPK), lambda i: (i, 0)),
        ],
        out_specs=pl.BlockSpec((tile, D), lambda i: (i, 0)),
        out_shape=jax.ShapeDtypeStruct((T, D), jnp.float32),
    )(ypair, ypair, topk_w)


def kernel(hidden_states, gate_w, expert_bias, w1, w3, w2):
    topk_idx, topk_w = _router(hidden_states, gate_w, expert_bias)
    be, total_nb, ppos = _dispatch_maps(topk_idx)
    pos = ppos.reshape(NW, TPW, TOPK).transpose(0, 2, 1)
    xs = _sc_dispatch(hidden_states, pos)
    ysc = _grouped_mlp(be, total_nb, xs, w1, w3, w2)
    pidx = jnp.concatenate([ppos[0::2], ppos[1::2]])
    ypair = _sc_gather_pairs(ysc, pidx.reshape(-1, GCH))
    return _pair_add(ypair, topk_w)


# speedup vs baseline: 1.1952x; 1.0645x over previous
import functools

import jax
import jax.numpy as jnp
from jax import lax
from jax.experimental import pallas as pl
from jax.experimental.pallas import tpu as pltpu
from jax.experimental.pallas import tpu_sc as plsc

E = 64
TOPK = 2
D = 1024
DFF = 512
T = 2048

ROWS = 128
NB = (T * TOPK) // ROWS + E
PAD = NB * ROWS

NC, NS = 2, 16
NW = NC * NS
GCH = 64
G_CHUNKS = PAD // (NW * GCH)
TPW = T // NW

_ROUTER_TILE = 256


def _router_body(x_ref, gw_ref, b_ref, idx_ref, w_ref, rank_ref, cnt_out_ref,
                 cnt_ref):
    x = x_ref[...]
    gw = gw_ref[...]
    logits = lax.dot_general(x, gw, (((1,), (1,)), ((), ())),
                             preferred_element_type=jnp.float32)
    scores = jax.nn.sigmoid(logits)
    biased = scores + b_ref[...]
    cols = lax.broadcasted_iota(jnp.int32, biased.shape, 1)
    m1 = jnp.max(biased, axis=1, keepdims=True)
    i1 = jnp.min(jnp.where(biased == m1, cols, E), axis=1, keepdims=True)
    oh1 = cols == i1
    s1 = jnp.sum(jnp.where(oh1, scores, 0.0), axis=1, keepdims=True)
    masked = jnp.where(oh1, -jnp.inf, biased)
    m2 = jnp.max(masked, axis=1, keepdims=True)
    i2 = jnp.min(jnp.where(masked == m2, cols, E), axis=1, keepdims=True)
    oh2 = cols == i2
    s2 = jnp.sum(jnp.where(oh2, scores, 0.0), axis=1, keepdims=True)
    tot = s1 + s2
    idx_ref[...] = jnp.concatenate([i1, i2], axis=1)
    w_ref[...] = jnp.concatenate([s1 / tot, s2 / tot], axis=1)

    @pl.when(pl.program_id(0) == 0)
    def _():
        cnt_ref[...] = jnp.zeros((1, E), jnp.float32)

    ohsum = oh1.astype(jnp.float32) + oh2.astype(jnp.float32)
    r = lax.broadcasted_iota(jnp.int32, (_ROUTER_TILE, _ROUTER_TILE), 0)
    c = lax.broadcasted_iota(jnp.int32, (_ROUTER_TILE, _ROUTER_TILE), 1)
    stri = (c < r).astype(jnp.float32)
    excl = jnp.dot(stri, ohsum, preferred_element_type=jnp.float32)
    base = excl + cnt_ref[...]
    r0 = jnp.sum(jnp.where(oh1, base, 0.0), axis=1, keepdims=True)
    r1 = jnp.sum(jnp.where(oh2, base, 0.0), axis=1, keepdims=True)
    rank_ref[...] = jnp.concatenate([r0, r1], axis=1).astype(jnp.int32)
    last = _ROUTER_TILE - 1
    new_cnt = base[last:last + 1, :] + ohsum[last:last + 1, :]
    cnt_ref[...] = new_cnt
    cnt_out_ref[...] = new_cnt.astype(jnp.int32)


def _router(x, gate_w, expert_bias):
    return pl.pallas_call(
        _router_body,
        grid=(T // _ROUTER_TILE,),
        in_specs=[
            pl.BlockSpec((_ROUTER_TILE, D), lambda i: (i, 0)),
            pl.BlockSpec((E, D), lambda i: (0, 0)),
            pl.BlockSpec((1, E), lambda i: (0, 0)),
        ],
        out_specs=[
            pl.BlockSpec((_ROUTER_TILE, TOPK), lambda i: (i, 0)),
            pl.BlockSpec((_ROUTER_TILE, TOPK), lambda i: (i, 0)),
            pl.BlockSpec((_ROUTER_TILE, TOPK), lambda i: (i, 0)),
            pl.BlockSpec((1, E), lambda i: (0, 0)),
        ],
        out_shape=[
            jax.ShapeDtypeStruct((T, TOPK), jnp.int32),
            jax.ShapeDtypeStruct((T, TOPK), jnp.float32),
            jax.ShapeDtypeStruct((T, TOPK), jnp.int32),
            jax.ShapeDtypeStruct((1, E), jnp.int32),
        ],
        scratch_shapes=[pltpu.VMEM((1, E), jnp.float32)],
        compiler_params=pltpu.CompilerParams(
            dimension_semantics=("arbitrary",)),
    )(x, gate_w, expert_bias.reshape(1, E))


def _dispatch_maps(topk_idx, rank, counts):
    eq = topk_idx.reshape(-1).astype(jnp.int32)
    rank = rank.reshape(-1)
    counts = counts.reshape(E)
    nb = (counts + ROWS - 1) // ROWS
    pcount = nb * ROWS
    poff = jnp.concatenate([jnp.zeros((1,), jnp.int32),
                            jnp.cumsum(pcount)[:-1].astype(jnp.int32)])
    ppos = poff[eq] + rank
    cnb = jnp.cumsum(nb).astype(jnp.int32)
    total_nb = cnb[-1]
    blk = jnp.arange(NB, dtype=jnp.int32)
    be_raw = jnp.clip(jnp.searchsorted(cnb, blk, side='right'), 0, E - 1)
    be_raw = be_raw.astype(jnp.int32)
    br_raw = poff[be_raw] // ROWS + (blk - (cnb[be_raw] - nb[be_raw]))
    valid = blk < total_nb
    be = jnp.where(valid, be_raw, be_raw[total_nb - 1])
    del br_raw
    return be, total_nb.reshape(1), ppos


def _dispatch_body(x_hbm, pos_hbm, xs_hbm, idx_v, rows_v, sem):
    wid = lax.axis_index("s") * NC + lax.axis_index("c")
    pltpu.sync_copy(pos_hbm.at[wid], idx_v)
    pltpu.sync_copy(x_hbm.at[pl.ds(wid * TPW, TPW)], rows_v)
    pltpu.async_copy(rows_v, xs_hbm.at[idx_v.at[0]], sem).wait()
    pltpu.async_copy(rows_v, xs_hbm.at[idx_v.at[1]], sem).wait()


_sc_dispatch = functools.partial(
    pl.kernel,
    mesh=plsc.VectorSubcoreMesh(core_axis_name="c", subcore_axis_name="s"),
    out_type=jax.ShapeDtypeStruct((PAD, D), jnp.float32),
    scratch_types=[
        pltpu.VMEM((TOPK, TPW), jnp.int32),
        pltpu.VMEM((TPW, D), jnp.float32),
        pltpu.SemaphoreType.DMA,
    ],
)(_dispatch_body)


def _make_sc_gather(n_rows):
    chunks = n_rows // (NW * GCH)

    def body(src_hbm, gidx_hbm, out_hbm, idx_v, rows_v, sem):
        wid = lax.axis_index("s") * NC + lax.axis_index("c")
        pltpu.sync_copy(gidx_hbm.at[pl.ds(wid * chunks, chunks)], idx_v)
        base = wid * (chunks * GCH)
        for c in range(chunks):
            pltpu.async_copy(src_hbm.at[idx_v.at[c]], rows_v, sem).wait()
            pltpu.sync_copy(rows_v, out_hbm.at[pl.ds(base + c * GCH, GCH)])

    return functools.partial(
        pl.kernel,
        mesh=plsc.VectorSubcoreMesh(core_axis_name="c", subcore_axis_name="s"),
        out_type=jax.ShapeDtypeStruct((n_rows, D), jnp.float32),
        scratch_types=[
            pltpu.VMEM((chunks, GCH), jnp.int32),
            pltpu.VMEM((GCH, D), jnp.float32),
            pltpu.SemaphoreType.DMA,
        ],
    )(body)


_sc_gather_pairs = _make_sc_gather(T * TOPK)


def _mlp_body(be_ref, tn_ref, xs_ref, w1_ref, w3_ref, w2_ref, out_ref):
    del be_ref

    @pl.when(pl.program_id(0) < tn_ref[0])
    def _():
        x = xs_ref[...]
        a = jnp.dot(x, w1_ref[0], preferred_element_type=jnp.float32)
        g = a * jax.nn.sigmoid(a) * jnp.dot(x, w3_ref[0],
                                            preferred_element_type=jnp.float32)
        out_ref[...] = jnp.dot(g, w2_ref[0], preferred_element_type=jnp.float32)


def _grouped_mlp(be, total_nb, xs, w1, w3, w2):
    grid_spec = pltpu.PrefetchScalarGridSpec(
        num_scalar_prefetch=2,
        grid=(NB,),
        in_specs=[
            pl.BlockSpec((ROWS, D), lambda i, be, tn: (i, 0)),
            pl.BlockSpec((1, D, DFF), lambda i, be, tn: (be[i], 0, 0)),
            pl.BlockSpec((1, D, DFF), lambda i, be, tn: (be[i], 0, 0)),
            pl.BlockSpec((1, DFF, D), lambda i, be, tn: (be[i], 0, 0)),
        ],
        out_specs=pl.BlockSpec((ROWS, D), lambda i, be, tn: (i, 0)),
    )
    return pl.pallas_call(
        _mlp_body,
        grid_spec=grid_spec,
        out_shape=jax.ShapeDtypeStruct((PAD, D), jnp.float32),
        compiler_params=pltpu.CompilerParams(
            dimension_semantics=("arbitrary",)),
    )(be, total_nb, xs, w1, w3, w2)


def _add_body(a_ref, b_ref, w_ref, o_ref):
    w = w_ref[...]
    o_ref[...] = a_ref[...] * w[:, 0:1] + b_ref[...] * w[:, 1:2]


def _pair_add(ypair, topk_w):
    tile = 256
    return pl.pallas_call(
        _add_body,
        grid=(T // tile,),
        in_specs=[
            pl.BlockSpec((tile, D), lambda i: (i, 0)),
            pl.BlockSpec((tile, D), lambda i: (i + T // tile, 0)),
            pl.BlockSpec((tile, TOPK), lambda i: (i, 0)),
        ],
        out_specs=pl.BlockSpec((tile, D), lambda i: (i, 0)),
        out_shape=jax.ShapeDtypeStruct((T, D), jnp.float32),
    )(ypair, ypair, topk_w)


def kernel(hidden_states, gate_w, expert_bias, w1, w3, w2):
    topk_idx, topk_w, rank, counts = _router(hidden_states, gate_w,
                                             expert_bias)
    be, total_nb, ppos = _dispatch_maps(topk_idx, rank, counts)
    pos = ppos.reshape(NW, TPW, TOPK).transpose(0, 2, 1)
    xs = _sc_dispatch(hidden_states, pos)
    ysc = _grouped_mlp(be, total_nb, xs, w1, w3, w2)
    pidx = jnp.concatenate([ppos[0::2], ppos[1::2]])
    ypair = _sc_gather_pairs(ysc, pidx.reshape(-1, GCH))
    return _pair_add(ypair, topk_w)


# speedup vs baseline: 1.1959x; 1.0006x over previous
import functools

import jax
import jax.numpy as jnp
from jax import lax
from jax.experimental import pallas as pl
from jax.experimental.pallas import tpu as pltpu
from jax.experimental.pallas import tpu_sc as plsc

E = 64
TOPK = 2
D = 1024
DFF = 512
T = 2048

ROWS = 128
NB = (T * TOPK) // ROWS + E
PAD = NB * ROWS

NC, NS = 2, 16
NW = NC * NS
GCH = 64
TPW = T // NW

_ROUTER_TILE = 256


def _router_body(x_ref, gw_ref, b_ref, idx_ref, w_ref, rank_ref, cnt_out_ref,
                 cnt_ref):
    x = x_ref[...]
    gw = gw_ref[...]
    logits = lax.dot_general(x, gw, (((1,), (1,)), ((), ())),
                             preferred_element_type=jnp.float32)
    scores = jax.nn.sigmoid(logits)
    biased = scores + b_ref[...]
    cols = lax.broadcasted_iota(jnp.int32, biased.shape, 1)
    m1 = jnp.max(biased, axis=1, keepdims=True)
    i1 = jnp.min(jnp.where(biased == m1, cols, E), axis=1, keepdims=True)
    oh1 = cols == i1
    s1 = jnp.sum(jnp.where(oh1, scores, 0.0), axis=1, keepdims=True)
    masked = jnp.where(oh1, -jnp.inf, biased)
    m2 = jnp.max(masked, axis=1, keepdims=True)
    i2 = jnp.min(jnp.where(masked == m2, cols, E), axis=1, keepdims=True)
    oh2 = cols == i2
    s2 = jnp.sum(jnp.where(oh2, scores, 0.0), axis=1, keepdims=True)
    tot = s1 + s2
    idx_ref[...] = jnp.concatenate([i1, i2], axis=1)
    w_ref[...] = jnp.concatenate([s1 / tot, s2 / tot], axis=1)

    @pl.when(pl.program_id(0) == 0)
    def _():
        cnt_ref[...] = jnp.zeros((1, E), jnp.float32)

    ohsum = oh1.astype(jnp.float32) + oh2.astype(jnp.float32)
    r = lax.broadcasted_iota(jnp.int32, (_ROUTER_TILE, _ROUTER_TILE), 0)
    c = lax.broadcasted_iota(jnp.int32, (_ROUTER_TILE, _ROUTER_TILE), 1)
    stri = (c < r).astype(jnp.float32)
    excl = jnp.dot(stri, ohsum, preferred_element_type=jnp.float32)
    base = excl + cnt_ref[...]
    r0 = jnp.sum(jnp.where(oh1, base, 0.0), axis=1, keepdims=True)
    r1 = jnp.sum(jnp.where(oh2, base, 0.0), axis=1, keepdims=True)
    rank_ref[...] = jnp.concatenate([r0, r1], axis=1).astype(jnp.int32)
    last = _ROUTER_TILE - 1
    new_cnt = base[last:last + 1, :] + ohsum[last:last + 1, :]
    cnt_ref[...] = new_cnt
    cnt_out_ref[...] = new_cnt.astype(jnp.int32)


def _router(x, gate_w, expert_bias):
    return pl.pallas_call(
        _router_body,
        grid=(T // _ROUTER_TILE,),
        in_specs=[
            pl.BlockSpec((_ROUTER_TILE, D), lambda i: (i, 0)),
            pl.BlockSpec((E, D), lambda i: (0, 0)),
            pl.BlockSpec((1, E), lambda i: (0, 0)),
        ],
        out_specs=[
            pl.BlockSpec((_ROUTER_TILE, TOPK), lambda i: (i, 0)),
            pl.BlockSpec((_ROUTER_TILE, TOPK), lambda i: (i, 0)),
            pl.BlockSpec((_ROUTER_TILE, TOPK), lambda i: (i, 0)),
            pl.BlockSpec((1, E), lambda i: (0, 0)),
        ],
        out_shape=[
            jax.ShapeDtypeStruct((T, TOPK), jnp.int32),
            jax.ShapeDtypeStruct((T, TOPK), jnp.float32),
            jax.ShapeDtypeStruct((T, TOPK), jnp.int32),
            jax.ShapeDtypeStruct((1, E), jnp.int32),
        ],
        scratch_shapes=[pltpu.VMEM((1, E), jnp.float32)],
        compiler_params=pltpu.CompilerParams(
            dimension_semantics=("arbitrary",)),
    )(x, gate_w, expert_bias.reshape(1, E))


def _dispatch_maps(topk_idx, rank, counts):
    eq = topk_idx.reshape(-1).astype(jnp.int32)
    rank = rank.reshape(-1)
    counts = counts.reshape(E)
    nb = (counts + ROWS - 1) // ROWS
    pcount = nb * ROWS
    poff = jnp.concatenate([jnp.zeros((1,), jnp.int32),
                            jnp.cumsum(pcount)[:-1].astype(jnp.int32)])
    ppos = poff[eq] + rank
    cnb = jnp.cumsum(nb).astype(jnp.int32)
    total_nb = cnb[-1]
    blk = jnp.arange(NB, dtype=jnp.int32)
    be_raw = jnp.clip(jnp.searchsorted(cnb, blk, side='right'), 0, E - 1)
    be_raw = be_raw.astype(jnp.int32)
    valid = blk < total_nb
    be = jnp.where(valid, be_raw, be_raw[total_nb - 1])
    return be, total_nb.reshape(1), ppos


def _dispatch_body(x_hbm, pos_hbm, xs_hbm, idx_v, rows_v, sem):
    wid = lax.axis_index("s") * NC + lax.axis_index("c")
    pltpu.sync_copy(pos_hbm.at[wid], idx_v)
    pltpu.sync_copy(x_hbm.at[pl.ds(wid * TPW, TPW)], rows_v)
    pltpu.async_copy(rows_v, xs_hbm.at[idx_v.at[0]], sem).wait()
    pltpu.async_copy(rows_v, xs_hbm.at[idx_v.at[1]], sem).wait()


_sc_dispatch = functools.partial(
    pl.kernel,
    mesh=plsc.VectorSubcoreMesh(core_axis_name="c", subcore_axis_name="s"),
    out_type=jax.ShapeDtypeStruct((PAD, D), jnp.float32),
    scratch_types=[
        pltpu.VMEM((TOPK, TPW), jnp.int32),
        pltpu.VMEM((TPW, D), jnp.float32),
        pltpu.SemaphoreType.DMA,
    ],
)(_dispatch_body)


def _make_sc_gather(n_rows):
    chunks = n_rows // (NW * GCH)

    def body(src_hbm, gidx_hbm, out_hbm, idx_v, rows_v, sem):
        wid = lax.axis_index("s") * NC + lax.axis_index("c")
        pltpu.sync_copy(gidx_hbm.at[pl.ds(wid * chunks, chunks)], idx_v)
        base = wid * (chunks * GCH)
        for c in range(chunks):
            pltpu.async_copy(src_hbm.at[idx_v.at[c]], rows_v, sem).wait()
            pltpu.sync_copy(rows_v, out_hbm.at[pl.ds(base + c * GCH, GCH)])

    return functools.partial(
        pl.kernel,
        mesh=plsc.VectorSubcoreMesh(core_axis_name="c", subcore_axis_name="s"),
        out_type=jax.ShapeDtypeStruct((n_rows, D), jnp.float32),
        scratch_types=[
            pltpu.VMEM((chunks, GCH), jnp.int32),
            pltpu.VMEM((GCH, D), jnp.float32),
            pltpu.SemaphoreType.DMA,
        ],
    )(body)


_sc_gather_pairs = _make_sc_gather(T * TOPK)


def _mlp_body(be_ref, tn_ref, xs_ref, w1_ref, w3_ref, w2_ref, out_ref):
    del be_ref

    @pl.when(pl.program_id(0) < tn_ref[0])
    def _():
        x = xs_ref[...]
        a = jnp.dot(x, w1_ref[0], preferred_element_type=jnp.float32)
        g = a * jax.nn.sigmoid(a) * jnp.dot(x, w3_ref[0],
                                            preferred_element_type=jnp.float32)
        out_ref[...] = jnp.dot(g, w2_ref[0], preferred_element_type=jnp.float32)


def _grouped_mlp(be, total_nb, xs, w1, w3, w2):
    grid_spec = pltpu.PrefetchScalarGridSpec(
        num_scalar_prefetch=2,
        grid=(NB,),
        in_specs=[
            pl.BlockSpec((ROWS, D), lambda i, be, tn: (i, 0)),
            pl.BlockSpec((1, D, DFF), lambda i, be, tn: (be[i], 0, 0)),
            pl.BlockSpec((1, D, DFF), lambda i, be, tn: (be[i], 0, 0)),
            pl.BlockSpec((1, DFF, D), lambda i, be, tn: (be[i], 0, 0)),
        ],
        out_specs=pl.BlockSpec((ROWS, D), lambda i, be, tn: (i, 0)),
    )
    return pl.pallas_call(
        _mlp_body,
        grid_spec=grid_spec,
        out_shape=jax.ShapeDtypeStruct((PAD, D), jnp.float32),
        compiler_params=pltpu.CompilerParams(
            dimension_semantics=("arbitrary",)),
    )(be, total_nb, xs, w1, w3, w2)


def _add_body(a_ref, b_ref, w_ref, o_ref):
    w = w_ref[...]
    o_ref[...] = a_ref[...] * w[:, 0:1] + b_ref[...] * w[:, 1:2]


def _pair_add(ypair, topk_w):
    tile = 256
    return pl.pallas_call(
        _add_body,
        grid=(T // tile,),
        in_specs=[
            pl.BlockSpec((tile, D), lambda i: (i, 0)),
            pl.BlockSpec((tile, D), lambda i: (i + T // tile, 0)),
            pl.BlockSpec((tile, TOPK), lambda i: (i, 0)),
        ],
        out_specs=pl.BlockSpec((tile, D), lambda i: (i, 0)),
        out_shape=jax.ShapeDtypeStruct((T, D), jnp.float32),
    )(ypair, ypair, topk_w)


def kernel(hidden_states, gate_w, expert_bias, w1, w3, w2):
    topk_idx, topk_w, rank, counts = _router(hidden_states, gate_w,
                                             expert_bias)
    be, total_nb, ppos = _dispatch_maps(topk_idx, rank, counts)
    pos = ppos.reshape(NW, TPW, TOPK).transpose(0, 2, 1)
    xs = _sc_dispatch(hidden_states, pos)
    ysc = _grouped_mlp(be, total_nb, xs, w1, w3, w2)
    pidx = jnp.concatenate([ppos[0::2], ppos[1::2]])
    ypair = _sc_gather_pairs(ysc, pidx.reshape(-1, GCH))
    return _pair_add(ypair, topk_w)
